# Initial kernel scaffold; baseline (speedup 1.0000x reference)
import jax
import jax.numpy as jnp
from jax.experimental import pallas as pl


def kernel(x, weight, bias):
    raise NotImplementedError("write your pallas kernel here")


# speedup vs baseline: 2.4651x; 2.4651x over previous
import jax
import jax.numpy as jnp
from jax.experimental import pallas as pl

_B = 8
_IC = 32
_OC = 32
_M = 96
_N = 96
_K = _M * (_N - 1) + _N * (_M - 1)
_OBLK = 4


def _deinterleave(w3):
    lo = w3[:, :, 0:128]
    hi = w3[:, :, 128:190]
    sh = w3.shape[:2]
    idx_lo = jax.lax.broadcasted_iota(jnp.int32, sh + (64,), 2) * 2
    idx_hi = jax.lax.broadcasted_iota(jnp.int32, sh + (31,), 2) * 2
    wa = jnp.concatenate(
        [jnp.take_along_axis(lo, idx_lo, axis=2),
         jnp.take_along_axis(hi, idx_hi, axis=2)], axis=2)
    wb = jnp.concatenate(
        [jnp.take_along_axis(lo, idx_lo + 1, axis=2),
         jnp.take_along_axis(hi, idx_hi + 1, axis=2)], axis=2)
    return wa, wb


def _body(xh_ref, xva_ref, xvb_ref, w_ref, b_ref, o_ref):
    s = pl.program_id(0)
    is_h = s < 12

    def compute(xa_of_ic, xb_of_ic):
        for o0 in range(0, _OC, _OBLK):
            bias = b_ref[...][0, o0:o0 + _OBLK]
            acc = jnp.broadcast_to(bias[None, :, None, None],
                                   (_B, _OBLK, 8, 95)).astype(jnp.float32)
            for ic in range(_IC):
                xa = xa_of_ic(ic)
                xb = xb_of_ic(ic)
                wa, wb = _deinterleave(w_ref[o0:o0 + _OBLK, ic, :, :])
                acc = acc + xa[:, None] * wa[None] + xb[:, None] * wb[None]
            o_ref[:, o0:o0 + _OBLK, :, :] = acc

    @pl.when(is_h)
    def _():
        compute(lambda ic: xh_ref[:, ic, :, 0:95],
                lambda ic: xh_ref[:, ic, :, 1:96])

    @pl.when(jnp.logical_not(is_h))
    def _():
        compute(lambda ic: xva_ref[:, ic, :, :],
                lambda ic: xvb_ref[:, ic, :, :])


def kernel(x, weight, bias):
    x_flat = x.reshape(_B, _IC, _M * _N)
    xva = x_flat[:, :, : 96 * 95].reshape(_B, _IC, 96, 95)
    xvb = x_flat[:, :, _N:].reshape(_B, _IC, 96, 95)
    w4 = weight.reshape(_OC, _IC, 192, 190)
    bias2 = bias.reshape(1, _OC)

    out = pl.pallas_call(
        _body,
        grid=(24,),
        in_specs=[
            pl.BlockSpec((_B, _IC, 8, 96), lambda s: (0, 0, jnp.minimum(s, 11), 0)),
            pl.BlockSpec((_B, _IC, 8, 95), lambda s: (0, 0, jnp.clip(s - 12, 0, 11), 0)),
            pl.BlockSpec((_B, _IC, 8, 95), lambda s: (0, 0, jnp.clip(s - 12, 0, 11), 0)),
            pl.BlockSpec((_OC, _IC, 8, 190), lambda s: (0, 0, s, 0)),
            pl.BlockSpec((1, _OC), lambda s: (0, 0)),
        ],
        out_specs=pl.BlockSpec((_B, _OC, 8, 95), lambda s: (0, 0, s, 0)),
        out_shape=jax.ShapeDtypeStruct((_B, _OC, 192, 95), jnp.float32),
    )(x, xva, xvb, w4, bias2)
    return out.reshape(_B, _OC, _K)


# speedup vs baseline: 4.9046x; 1.9896x over previous
import jax
import jax.numpy as jnp
from jax.experimental import pallas as pl
from jax.experimental.pallas import tpu as pltpu

_B = 8
_IC = 32
_OC = 32
_M = 96
_N = 96
_K = _M * (_N - 1) + _N * (_M - 1)
_OBLK = 4


def _body(xh_ref, xva_ref, xvb_ref, w_ref, b_ref, o_ref, wa_s, wb_s):
    s = pl.program_id(0)
    is_h = s < 12

    idx_lo = jax.lax.broadcasted_iota(jnp.int32, (_IC, 8, 64), 2) * 2
    idx_hi = jax.lax.broadcasted_iota(jnp.int32, (_IC, 8, 31), 2) * 2
    for o in range(_OC):
        lo = w_ref[o, :, :, 0:128]
        wa_s[o, :, :, 0:64] = jnp.take_along_axis(lo, idx_lo, axis=2)
    for o in range(_OC):
        lo = w_ref[o, :, :, 0:128]
        wb_s[o, :, :, 0:64] = jnp.take_along_axis(lo, idx_lo + 1, axis=2)
    for o in range(_OC):
        hi = w_ref[o, :, :, 128:190]
        wa_s[o, :, :, 64:95] = jnp.take_along_axis(hi, idx_hi, axis=2)
    for o in range(_OC):
        hi = w_ref[o, :, :, 128:190]
        wb_s[o, :, :, 64:95] = jnp.take_along_axis(hi, idx_hi + 1, axis=2)

    def compute(xa_of, xb_of):
        for o0 in range(0, _OC, _OBLK):
            bias = b_ref[...][0, o0:o0 + _OBLK]
            acc0 = jnp.broadcast_to(bias[:, None, None],
                                    (_OBLK, 8, 95)).astype(jnp.float32)
            accs = [acc0] * _B
            for ic in range(_IC):
                wa = wa_s[o0:o0 + _OBLK, ic, :, :]
                wb = wb_s[o0:o0 + _OBLK, ic, :, :]
                for b in range(_B):
                    xa = xa_of(b, ic)
                    xb = xb_of(b, ic)
                    accs[b] = accs[b] + xa[None] * wa + xb[None] * wb
            for b in range(_B):
                o_ref[b, o0:o0 + _OBLK, :, :] = accs[b]

    @pl.when(is_h)
    def _():
        compute(lambda b, ic: xh_ref[b, ic, :, 0:95],
                lambda b, ic: xh_ref[b, ic, :, 1:96])

    @pl.when(jnp.logical_not(is_h))
    def _():
        compute(lambda b, ic: xva_ref[b, ic, :, :],
                lambda b, ic: xvb_ref[b, ic, :, :])


def kernel(x, weight, bias):
    x_flat = x.reshape(_B, _IC, _M * _N)
    xva = x_flat[:, :, : 96 * 95].reshape(_B, _IC, 96, 95)
    xvb = x_flat[:, :, _N:].reshape(_B, _IC, 96, 95)
    w4 = weight.reshape(_OC, _IC, 192, 190)
    bias2 = bias.reshape(1, _OC)

    out = pl.pallas_call(
        _body,
        grid=(24,),
        in_specs=[
            pl.BlockSpec((_B, _IC, 8, 96), lambda s: (0, 0, jnp.minimum(s, 11), 0)),
            pl.BlockSpec((_B, _IC, 8, 95), lambda s: (0, 0, jnp.clip(s - 12, 0, 11), 0)),
            pl.BlockSpec((_B, _IC, 8, 95), lambda s: (0, 0, jnp.clip(s - 12, 0, 11), 0)),
            pl.BlockSpec((_OC, _IC, 8, 190), lambda s: (0, 0, s, 0)),
            pl.BlockSpec((1, _OC), lambda s: (0, 0)),
        ],
        out_specs=pl.BlockSpec((_B, _OC, 8, 95), lambda s: (0, 0, s, 0)),
        out_shape=jax.ShapeDtypeStruct((_B, _OC, 192, 95), jnp.float32),
        scratch_shapes=[
            pltpu.VMEM((_OC, _IC, 8, 95), jnp.float32),
            pltpu.VMEM((_OC, _IC, 8, 95), jnp.float32),
        ],
    )(x, xva, xvb, w4, bias2)
    return out.reshape(_B, _OC, _K)
